# Initial kernel scaffold; baseline (speedup 1.0000x reference)
import jax
import jax.numpy as jnp
from jax.experimental import pallas as pl


def kernel(previous_embedding, token_type_ids, token_type_table):
    raise NotImplementedError("write your pallas kernel here")


# speedup vs baseline: 1.7606x; 1.7606x over previous
import jax
import jax.numpy as jnp
from jax.experimental import pallas as pl

_BLK = 2048


def _blend_kernel(ids_ref, prev_ref, tab_ref, out_ref):
    t0 = tab_ref[0, :]
    t1 = tab_ref[1, :]
    sel = ids_ref[...]
    out_ref[...] = prev_ref[...] + (t0 + sel * (t1 - t0))


def kernel(previous_embedding, token_type_ids, token_type_table):
    b, s, w = previous_embedding.shape
    n = b * s
    prev = previous_embedding.reshape(n, w)
    ids = token_type_ids.reshape(n, 1).astype(jnp.float32)
    out = pl.pallas_call(
        _blend_kernel,
        grid=(n // _BLK,),
        in_specs=[
            pl.BlockSpec((_BLK, 1), lambda i: (i, 0)),
            pl.BlockSpec((_BLK, w), lambda i: (i, 0)),
            pl.BlockSpec((2, w), lambda i: (0, 0)),
        ],
        out_specs=pl.BlockSpec((_BLK, w), lambda i: (i, 0)),
        out_shape=jax.ShapeDtypeStruct((n, w), jnp.float32),
    )(ids, prev, token_type_table)
    return out.reshape(b, s, w)
